# baseline (speedup 1.0000x reference)
import functools

import jax
import jax.numpy as jnp
from jax import lax
from jax.experimental import pallas as pl
from jax.experimental.pallas import tpu as pltpu
from jax.experimental.pallas import tpu_sc as plsc


def _gather_sc(idx_flat, emb):
    N = idx_flat.shape[0]
    E = emb.shape[1]
    info = plsc.get_sparse_core_info()
    nw = info.num_cores * info.num_subcores
    per_w = N // nw
    ch = 80
    nch = per_w // ch
    assert per_w % ch == 0 and N % nw == 0

    mesh = plsc.VectorSubcoreMesh(core_axis_name="c", subcore_axis_name="s")

    @functools.partial(
        pl.kernel,
        mesh=mesh,
        out_type=jax.ShapeDtypeStruct((N, E), jnp.float32),
        scratch_types=[
            pltpu.VMEM((ch,), jnp.int32),
            pltpu.VMEM((ch, E), jnp.float32),
            pltpu.SemaphoreType.DMA,
        ],
    )
    def gk(idx_hbm, emb_hbm, out_hbm, idx_v, rows_v, sem):
        wid = lax.axis_index("s") * info.num_cores + lax.axis_index("c")
        base = wid * per_w

        def chunk(j, carry):
            off = base + j * ch
            pltpu.sync_copy(idx_hbm.at[pl.ds(off, ch)], idx_v)
            pltpu.async_copy(emb_hbm.at[idx_v], rows_v, sem).wait()
            pltpu.sync_copy(rows_v, out_hbm.at[pl.ds(off, ch)])
            return carry

        lax.fori_loop(0, nch, chunk, 0)

    return gk(idx_flat, emb)


def _sig(x):
    return 0.5 * jnp.tanh(0.5 * x) + 0.5


def _lstm_body(e_ref, wcat_ref, b_ref, wc_ref, bc_ref, out_ref,
               xh_scr, c_scr):
    t = pl.program_id(0)
    T = pl.num_programs(0)
    E = e_ref.shape[2]
    H = c_scr.shape[1]

    @pl.when(t == 0)
    def _init():
        xh_scr[:, E:] = jnp.zeros_like(xh_scr[:, E:])
        c_scr[...] = jnp.zeros_like(c_scr)

    xh_scr[:, :E] = e_ref[0].astype(xh_scr.dtype)
    gates = jnp.dot(xh_scr[...], wcat_ref[...],
                    preferred_element_type=jnp.float32).astype(jnp.bfloat16)
    b = b_ref[...]

    def gsl(k):
        return gates[:, k * H:(k + 1) * H].astype(jnp.float32)

    ig = _sig(gsl(0) + b[:, :H])
    fg = _sig(gsl(1) + b[:, H:2 * H])
    gg = jnp.tanh(gsl(2) + b[:, 2 * H:3 * H])
    og = _sig(gsl(3) + b[:, 3 * H:])
    c = fg * c_scr[...] + ig * gg
    c_scr[...] = c
    xh_scr[:, E:] = (og * jnp.tanh(c)).astype(xh_scr.dtype)

    @pl.when(t == T - 1)
    def _cls():
        out_ref[...] = (
            jnp.dot(xh_scr[:, E:], wc_ref[...],
                    preferred_element_type=jnp.float32)
            + bc_ref[...]
        )


def _lstm_tc(e_tm, wcat, bias, wc, bc):
    T, B, E = e_tm.shape
    H = wc.shape[0]
    return pl.pallas_call(
        _lstm_body,
        grid=(T,),
        in_specs=[
            pl.BlockSpec((1, B, E), lambda t: (t, 0, 0)),
            pl.BlockSpec((E + H, 4 * H), lambda t: (0, 0)),
            pl.BlockSpec((1, 4 * H), lambda t: (0, 0)),
            pl.BlockSpec((H, 128), lambda t: (0, 0)),
            pl.BlockSpec((1, 128), lambda t: (0, 0)),
        ],
        out_specs=pl.BlockSpec((B, 128), lambda t: (0, 0)),
        out_shape=jax.ShapeDtypeStruct((B, 128), jnp.float32),
        scratch_shapes=[
            pltpu.VMEM((B, E + H), jnp.bfloat16),
            pltpu.VMEM((B, H), jnp.float32),
        ],
    )(e_tm, wcat.astype(jnp.bfloat16), bias, wc.astype(jnp.bfloat16), bc)


def kernel(x, emb, W_ih, W_hh, b_ih, b_hh, W_cls, b_cls):
    B, T = x.shape
    E = emb.shape[1]
    H = W_hh.shape[1]
    ncls = W_cls.shape[0]

    idx_tm = x.T.reshape(-1).astype(jnp.int32)
    e_flat = _gather_sc(idx_tm, emb.astype(jnp.float32))
    e_tm = e_flat.reshape(T, B, E)

    wcat = jnp.concatenate([W_ih.T, W_hh.T], axis=0)
    bias = (b_ih + b_hh).reshape(1, 4 * H)
    wc = jnp.zeros((H, 128), jnp.float32).at[:, :ncls].set(W_cls.T)
    bc = jnp.zeros((1, 128), jnp.float32).at[:, :ncls].set(b_cls)

    out = _lstm_tc(e_tm, wcat, bias, wc, bc)
    return out[:, :ncls]


# speedup vs baseline: 1.1254x; 1.1254x over previous
import functools

import jax
import jax.numpy as jnp
from jax import lax
from jax.experimental import pallas as pl
from jax.experimental.pallas import tpu as pltpu
from jax.experimental.pallas import tpu_sc as plsc


def _gather_sc(idx_flat, emb):
    N = idx_flat.shape[0]
    E = emb.shape[1]
    info = plsc.get_sparse_core_info()
    nw = info.num_cores * info.num_subcores
    per_w = N // nw
    ch = 80
    nch = per_w // ch
    assert per_w % ch == 0 and N % nw == 0

    idx3 = idx_flat.reshape(nw, nch, ch)
    mesh = plsc.VectorSubcoreMesh(core_axis_name="c", subcore_axis_name="s")

    @functools.partial(
        pl.kernel,
        mesh=mesh,
        out_type=jax.ShapeDtypeStruct((N, E), jnp.float32),
        scratch_types=[
            pltpu.VMEM((nch, ch), jnp.int32),
            pltpu.VMEM((ch, E), jnp.float32),
            pltpu.VMEM((ch, E), jnp.float32),
            pltpu.SemaphoreType.DMA,
            pltpu.SemaphoreType.DMA,
            pltpu.SemaphoreType.DMA,
            pltpu.SemaphoreType.DMA,
        ],
    )
    def gk(idx_hbm, emb_hbm, out_hbm, idx_v, rows_a, rows_b,
           ga_sem, gb_sem, sa_sem, sb_sem):
        wid = lax.axis_index("s") * info.num_cores + lax.axis_index("c")
        base = wid * per_w
        pltpu.sync_copy(idx_hbm.at[wid], idx_v)
        bufs = (rows_a, rows_b)
        gsems = (ga_sem, gb_sem)
        ssems = (sa_sem, sb_sem)
        gh = [None] * nch
        sh = [None] * nch
        for j in range(nch):
            if j >= 2:
                sh[j - 2].wait()
            gh[j] = pltpu.async_copy(emb_hbm.at[idx_v.at[j]], bufs[j % 2],
                                     gsems[j % 2])
            if j >= 1:
                gh[j - 1].wait()
                sh[j - 1] = pltpu.async_copy(
                    bufs[(j - 1) % 2],
                    out_hbm.at[pl.ds(base + (j - 1) * ch, ch)],
                    ssems[(j - 1) % 2])
        gh[nch - 1].wait()
        sh[nch - 1] = pltpu.async_copy(
            bufs[(nch - 1) % 2],
            out_hbm.at[pl.ds(base + (nch - 1) * ch, ch)],
            ssems[(nch - 1) % 2])
        sh[nch - 2].wait()
        sh[nch - 1].wait()

    return gk(idx3, emb)


def _sig(x):
    return 0.5 * jnp.tanh(0.5 * x) + 0.5


def _lstm_body(e_ref, wcat_ref, b_ref, wc_ref, bc_ref, out_ref,
               xh_scr, c_scr):
    t = pl.program_id(0)
    T = pl.num_programs(0)
    E = e_ref.shape[2]
    H = c_scr.shape[1]

    @pl.when(t == 0)
    def _init():
        xh_scr[:, E:] = jnp.zeros_like(xh_scr[:, E:])
        c_scr[...] = jnp.zeros_like(c_scr)

    xh_scr[:, :E] = e_ref[0].astype(xh_scr.dtype)
    gates = jnp.dot(xh_scr[...], wcat_ref[...],
                    preferred_element_type=jnp.float32)
    b = b_ref[...]
    ig = _sig(gates[:, :H] + b[:, :H])
    fg = _sig(gates[:, H:2 * H] + b[:, H:2 * H])
    gg = jnp.tanh(gates[:, 2 * H:3 * H] + b[:, 2 * H:3 * H])
    og = _sig(gates[:, 3 * H:] + b[:, 3 * H:])
    c = fg * c_scr[...] + ig * gg
    c_scr[...] = c
    xh_scr[:, E:] = (og * jnp.tanh(c)).astype(xh_scr.dtype)

    @pl.when(t == T - 1)
    def _cls():
        out_ref[...] = (
            jnp.dot(xh_scr[:, E:], wc_ref[...],
                    preferred_element_type=jnp.float32)
            + bc_ref[...]
        )


def _lstm_tc(e_tm, wcat, bias, wc, bc):
    T, B, E = e_tm.shape
    H = wc.shape[0]
    return pl.pallas_call(
        _lstm_body,
        grid=(T,),
        in_specs=[
            pl.BlockSpec((1, B, E), lambda t: (t, 0, 0)),
            pl.BlockSpec((E + H, 4 * H), lambda t: (0, 0)),
            pl.BlockSpec((1, 4 * H), lambda t: (0, 0)),
            pl.BlockSpec((H, 128), lambda t: (0, 0)),
            pl.BlockSpec((1, 128), lambda t: (0, 0)),
        ],
        out_specs=pl.BlockSpec((B, 128), lambda t: (0, 0)),
        out_shape=jax.ShapeDtypeStruct((B, 128), jnp.float32),
        scratch_shapes=[
            pltpu.VMEM((B, E + H), jnp.bfloat16),
            pltpu.VMEM((B, H), jnp.float32),
        ],
    )(e_tm, wcat.astype(jnp.bfloat16), bias, wc.astype(jnp.bfloat16), bc)


def kernel(x, emb, W_ih, W_hh, b_ih, b_hh, W_cls, b_cls):
    B, T = x.shape
    E = emb.shape[1]
    H = W_hh.shape[1]
    ncls = W_cls.shape[0]

    idx_tm = x.T.reshape(-1).astype(jnp.int32)
    e_flat = _gather_sc(idx_tm, emb.astype(jnp.float32))
    e_tm = e_flat.reshape(T, B, E)

    wcat = jnp.concatenate([W_ih.T, W_hh.T], axis=0)
    bias = (b_ih + b_hh).reshape(1, 4 * H)
    wc = jnp.zeros((H, 128), jnp.float32).at[:, :ncls].set(W_cls.T)
    bc = jnp.zeros((1, 128), jnp.float32).at[:, :ncls].set(b_cls)

    out = _lstm_tc(e_tm, wcat, bias, wc, bc)
    return out[:, :ncls]


# speedup vs baseline: 1.1377x; 1.0109x over previous
import functools

import jax
import jax.numpy as jnp
from jax import lax
from jax.experimental import pallas as pl
from jax.experimental.pallas import tpu as pltpu
from jax.experimental.pallas import tpu_sc as plsc


def _gather_sc(idx_flat, emb):
    N = idx_flat.shape[0]
    E = emb.shape[1]
    info = plsc.get_sparse_core_info()
    nw = info.num_cores * info.num_subcores
    per_w = N // nw
    ch = 80
    nch = per_w // ch
    assert per_w % ch == 0 and N % nw == 0

    idx3 = idx_flat.reshape(nw, nch, ch)
    mesh = plsc.VectorSubcoreMesh(core_axis_name="c", subcore_axis_name="s")

    @functools.partial(
        pl.kernel,
        mesh=mesh,
        out_type=jax.ShapeDtypeStruct((N, E), jnp.float32),
        scratch_types=(
            [pltpu.VMEM((nch, ch), jnp.int32)]
            + [pltpu.VMEM((ch, E), jnp.float32) for _ in range(4)]
            + [pltpu.SemaphoreType.DMA for _ in range(8)]
        ),
    )
    def gk(idx_hbm, emb_hbm, out_hbm, idx_v, *rs):
        bufs = rs[:4]
        gsems = rs[4:8]
        ssems = rs[8:12]
        wid = lax.axis_index("s") * info.num_cores + lax.axis_index("c")
        base = wid * per_w
        pltpu.sync_copy(idx_hbm.at[wid], idx_v)
        gh = [None] * nch
        sh = [None] * nch

        def store(j):
            gh[j].wait()
            sh[j] = pltpu.async_copy(
                bufs[j % 4], out_hbm.at[pl.ds(base + j * ch, ch)],
                ssems[j % 4])

        for j in range(nch):
            if j >= 4:
                sh[j - 4].wait()
            gh[j] = pltpu.async_copy(emb_hbm.at[idx_v.at[j]], bufs[j % 4],
                                     gsems[j % 4])
            if j >= 2:
                store(j - 2)
        store(nch - 2)
        store(nch - 1)
        for j in range(nch - 4, nch):
            sh[j].wait()

    return gk(idx3, emb)


def _sig(x):
    return 0.5 * jnp.tanh(0.5 * x) + 0.5


def _lstm_body(e_ref, wcat_ref, b_ref, wc_ref, bc_ref, out_ref,
               xh_scr, c_scr):
    t = pl.program_id(0)
    T = pl.num_programs(0)
    E = e_ref.shape[2]
    H = c_scr.shape[1]

    @pl.when(t == 0)
    def _init():
        xh_scr[:, E:] = jnp.zeros_like(xh_scr[:, E:])
        c_scr[...] = jnp.zeros_like(c_scr)

    xh_scr[:, :E] = e_ref[0].astype(xh_scr.dtype)
    gates = jnp.dot(xh_scr[...], wcat_ref[...],
                    preferred_element_type=jnp.float32)
    b = b_ref[...]
    ig = _sig(gates[:, :H] + b[:, :H])
    fg = _sig(gates[:, H:2 * H] + b[:, H:2 * H])
    gg = jnp.tanh(gates[:, 2 * H:3 * H] + b[:, 2 * H:3 * H])
    og = _sig(gates[:, 3 * H:] + b[:, 3 * H:])
    c = fg * c_scr[...] + ig * gg
    c_scr[...] = c
    xh_scr[:, E:] = (og * jnp.tanh(c)).astype(xh_scr.dtype)

    @pl.when(t == T - 1)
    def _cls():
        out_ref[...] = (
            jnp.dot(xh_scr[:, E:], wc_ref[...],
                    preferred_element_type=jnp.float32)
            + bc_ref[...]
        )


def _lstm_tc(e_tm, wcat, bias, wc, bc):
    T, B, E = e_tm.shape
    H = wc.shape[0]
    return pl.pallas_call(
        _lstm_body,
        grid=(T,),
        in_specs=[
            pl.BlockSpec((1, B, E), lambda t: (t, 0, 0)),
            pl.BlockSpec((E + H, 4 * H), lambda t: (0, 0)),
            pl.BlockSpec((1, 4 * H), lambda t: (0, 0)),
            pl.BlockSpec((H, 128), lambda t: (0, 0)),
            pl.BlockSpec((1, 128), lambda t: (0, 0)),
        ],
        out_specs=pl.BlockSpec((B, 128), lambda t: (0, 0)),
        out_shape=jax.ShapeDtypeStruct((B, 128), jnp.float32),
        scratch_shapes=[
            pltpu.VMEM((B, E + H), jnp.bfloat16),
            pltpu.VMEM((B, H), jnp.float32),
        ],
    )(e_tm, wcat.astype(jnp.bfloat16), bias, wc.astype(jnp.bfloat16), bc)


def kernel(x, emb, W_ih, W_hh, b_ih, b_hh, W_cls, b_cls):
    B, T = x.shape
    E = emb.shape[1]
    H = W_hh.shape[1]
    ncls = W_cls.shape[0]

    idx_tm = x.T.reshape(-1).astype(jnp.int32)
    e_flat = _gather_sc(idx_tm, emb.astype(jnp.float32))
    e_tm = e_flat.reshape(T, B, E)

    wcat = jnp.concatenate([W_ih.T, W_hh.T], axis=0)
    bias = (b_ih + b_hh).reshape(1, 4 * H)
    wc = jnp.zeros((H, 128), jnp.float32).at[:, :ncls].set(W_cls.T)
    bc = jnp.zeros((1, 128), jnp.float32).at[:, :ncls].set(b_cls)

    out = _lstm_tc(e_tm, wcat, bias, wc, bc)
    return out[:, :ncls]
